# baseline (speedup 1.0000x reference)
import functools

import jax
import jax.numpy as jnp
from jax import lax
from jax.experimental import pallas as pl
from jax.experimental.pallas import tpu as pltpu
from jax.experimental.pallas import tpu_sc as plsc

N = 10000
E = 160000
IN_DIM = 256
OUT_DIM = 256
NUM_HEADS = 4
HEAD_DIM = OUT_DIM // NUM_HEADS
EPS = 1e-5

NC = 2
NS = 16
L = 16
HALF = OUT_DIM // NC
EPT = E // NS
K = 64
NCHUNK = 160
EPT_P = NCHUNK * K
BLK = 16
NBLK = NCHUNK // BLK
ACC_ROWS = 10240
RPT = ACC_ROWS // NS
PAD_DST = ACC_ROWS - 1


def _mm_body(x_ref, w_ref, o_ref):
    o_ref[0] = jnp.dot(x_ref[...], w_ref[...],
                       preferred_element_type=jnp.float32)


def _matmul_split(x, w_cat):
    RB = 2000
    return pl.pallas_call(
        _mm_body,
        grid=(N // RB, NC),
        in_specs=[
            pl.BlockSpec((RB, IN_DIM), lambda i, j: (i, 0)),
            pl.BlockSpec((IN_DIM, HALF), lambda i, j: (0, j)),
        ],
        out_specs=pl.BlockSpec((1, RB, HALF), lambda i, j: (j, i, 0)),
        out_shape=jax.ShapeDtypeStruct((NC, N, HALF), jnp.float32),
    )(x, w_cat)


def _sc_body(h_hbm, src_hbm, dst_hbm, val_hbm, zeros_hbm, out_hbm,
             srcA, srcB, dstA, dstB, valA, valB,
             rows0, rows1, rows2, rows3, acc,
             gsem0, gsem1, gsem2, gsem3, ssem0, ssem1, ssem2, ssem3):
    c = lax.axis_index("c")
    s = lax.axis_index("s")
    hview = h_hbm.at[c]

    def load_block(b, srcb, dstb, valb):
        pltpu.sync_copy(src_hbm.at[s].at[pl.ds(b * BLK, BLK)], srcb)
        pltpu.sync_copy(dst_hbm.at[s].at[pl.ds(b * BLK, BLK)], dstb)
        pltpu.sync_copy(val_hbm.at[s].at[pl.ds(b * BLK, BLK)], valb)

    def gather_start(srcb, j, rows, sem):
        pltpu.async_copy(hview.at[srcb.at[j]], rows, sem)

    def gather_wait(rows, sem):
        pltpu.make_async_copy(hview.at[srcA.at[0]], rows, sem).wait()

    def scat_start(dstb, j, rows, sem):
        pltpu.async_copy(rows, acc.at[dstb.at[j]], sem, add=True)

    def scat_wait(dstb, rows, sem):
        pltpu.make_async_copy(rows, acc.at[dstb.at[0]], sem).wait()

    def scale(valb, j, rows):
        def group(g, cc):
            vv = valb[j, pl.ds(g * L, L)]
            for i in range(L):
                vsp = jnp.full((L,), vv[i], jnp.float32)
                for ci in range(HALF // L):
                    sl = pl.ds(ci * L, L)
                    rows[g * L + i, sl] = rows[g * L + i, sl] * vsp
            return cc

        lax.fori_loop(0, K // L, group, 0)

    bufs = [(srcA, dstA, valA), (srcB, dstB, valB)]
    R = [(rows0, gsem0, ssem0), (rows1, gsem1, ssem1),
         (rows2, gsem2, ssem2), (rows3, gsem3, ssem3)]
    load_block(0, *bufs[0])
    gather_start(bufs[0][0], 0, R[0][0], R[0][1])
    gather_start(bufs[0][0], 1, R[1][0], R[1][1])

    pltpu.sync_copy(zeros_hbm, acc.at[pl.ds(s * RPT, RPT)])
    plsc.subcore_barrier()

    def process(dstb, valb, t, rbuf, gsem, ssem):
        gather_wait(rbuf, gsem)
        scale(valb, t, rbuf)
        scat_start(dstb, t, rbuf, ssem)

    for b in range(NBLK):
        srcb, dstb, valb = bufs[b % 2]

        def quad(q, cc, srcb=srcb, dstb=dstb, valb=valb, first=(b == 0)):
            t0 = 4 * q

            def wait_tail():
                scat_wait(dstb, R[2][0], R[2][2])
                scat_wait(dstb, R[3][0], R[3][2])

            if first:
                pl.when(q > 0)(wait_tail)
            else:
                wait_tail()

            gather_start(srcb, t0 + 2, R[2][0], R[2][1])
            gather_start(srcb, t0 + 3, R[3][0], R[3][1])
            process(dstb, valb, t0, *R[0])
            process(dstb, valb, t0 + 1, *R[1])
            scat_wait(dstb, R[0][0], R[0][2])
            scat_wait(dstb, R[1][0], R[1][2])

            @pl.when(q < BLK // 4 - 1)
            def _():
                gather_start(srcb, t0 + 4, R[0][0], R[0][1])
                gather_start(srcb, t0 + 5, R[1][0], R[1][1])

            process(dstb, valb, t0 + 2, *R[2])
            process(dstb, valb, t0 + 3, *R[3])
            return cc

        lax.fori_loop(0, BLK // 4, quad, 0)

        if b + 1 < NBLK:
            load_block(b + 1, *bufs[(b + 1) % 2])
            nsrcb = bufs[(b + 1) % 2][0]
            gather_start(nsrcb, 0, R[0][0], R[0][1])
            gather_start(nsrcb, 1, R[1][0], R[1][1])

    scat_wait(bufs[(NBLK - 1) % 2][1], R[2][0], R[2][2])
    scat_wait(bufs[(NBLK - 1) % 2][1], R[3][0], R[3][2])
    plsc.subcore_barrier()

    pltpu.sync_copy(acc.at[pl.ds(s * RPT, RPT)],
                    out_hbm.at[c].at[pl.ds(s * RPT, RPT)])


def _sc_scatter(h_split, src, dst, val, zeros):
    mesh = plsc.VectorSubcoreMesh(core_axis_name="c", subcore_axis_name="s")
    return pl.kernel(
        _sc_body,
        out_type=jax.ShapeDtypeStruct((NC, ACC_ROWS, HALF), jnp.float32),
        mesh=mesh,
        scratch_types=[
            pltpu.VMEM((BLK, K), jnp.int32),
            pltpu.VMEM((BLK, K), jnp.int32),
            pltpu.VMEM((BLK, K), jnp.int32),
            pltpu.VMEM((BLK, K), jnp.int32),
            pltpu.VMEM((BLK, K), jnp.float32),
            pltpu.VMEM((BLK, K), jnp.float32),
            pltpu.VMEM((K, HALF), jnp.float32),
            pltpu.VMEM((K, HALF), jnp.float32),
            pltpu.VMEM((K, HALF), jnp.float32),
            pltpu.VMEM((K, HALF), jnp.float32),
            pltpu.VMEM_SHARED((ACC_ROWS, HALF), jnp.float32),
            pltpu.SemaphoreType.DMA,
            pltpu.SemaphoreType.DMA,
            pltpu.SemaphoreType.DMA,
            pltpu.SemaphoreType.DMA,
            pltpu.SemaphoreType.DMA,
            pltpu.SemaphoreType.DMA,
            pltpu.SemaphoreType.DMA,
            pltpu.SemaphoreType.DMA,
        ],
    )(h_split, src, dst, val, zeros)


def _ln_rows(a, g_row, b_row):
    mu = jnp.mean(a, axis=-1, keepdims=True)
    xc = a - mu
    var = jnp.mean(xc * xc, axis=-1, keepdims=True)
    return xc * lax.rsqrt(var + EPS) * g_row + b_row


def _ep_body(agg_ref, bias_ref, lng_ref, lnb_ref, outg_ref, outb_ref, o_ref):
    full = jnp.concatenate([agg_ref[0], agg_ref[1]], axis=-1)
    full = full + bias_ref[...]
    heads = []
    for h in range(NUM_HEADS):
        sl = slice(h * HEAD_DIM, (h + 1) * HEAD_DIM)
        heads.append(_ln_rows(full[:, sl], lng_ref[:, sl], lnb_ref[:, sl]))
    normed = jnp.concatenate(heads, axis=-1)
    o_ref[...] = _ln_rows(normed, outg_ref[...], outb_ref[...])


def _epilogue(agg_split, bias_row, lng_row, lnb_row, outg_row, outb_row):
    RB = 2000
    row_spec = pl.BlockSpec((1, OUT_DIM), lambda i: (0, 0))
    return pl.pallas_call(
        _ep_body,
        grid=(N // RB,),
        in_specs=[
            pl.BlockSpec((NC, RB, HALF), lambda i: (0, i, 0)),
            row_spec, row_spec, row_spec, row_spec, row_spec,
        ],
        out_specs=pl.BlockSpec((RB, OUT_DIM), lambda i: (i, 0)),
        out_shape=jax.ShapeDtypeStruct((N, OUT_DIM), jnp.float32),
    )(agg_split, bias_row, lng_row, lnb_row, outg_row, outb_row)


def kernel(x, adj_indices, adj_values, W, bias, ln_g, ln_b, out_g, out_b):
    w_cat = W.transpose(1, 0, 2).reshape(IN_DIM, OUT_DIM)
    h_split = _matmul_split(x, w_cat)

    pad = EPT_P - EPT
    dst = jnp.concatenate(
        [adj_indices[0].reshape(NS, EPT),
         jnp.full((NS, pad), PAD_DST, jnp.int32)], axis=1).reshape(NS, NCHUNK, K)
    src = jnp.concatenate(
        [adj_indices[1].reshape(NS, EPT),
         jnp.zeros((NS, pad), jnp.int32)], axis=1).reshape(NS, NCHUNK, K)
    val = jnp.concatenate(
        [adj_values.reshape(NS, EPT),
         jnp.zeros((NS, pad), jnp.float32)], axis=1).reshape(NS, NCHUNK, K)
    zeros = jnp.zeros((RPT, HALF), jnp.float32)
    agg_split = _sc_scatter(h_split, src, dst, val, zeros)

    bias_row = bias.reshape(1, OUT_DIM)
    lng_row = ln_g.reshape(1, OUT_DIM)
    lnb_row = ln_b.reshape(1, OUT_DIM)
    outg_row = out_g.reshape(1, OUT_DIM)
    outb_row = out_b.reshape(1, OUT_DIM)
    return _epilogue(agg_split, bias_row, lng_row, lnb_row, outg_row, outb_row)


# speedup vs baseline: 1.0294x; 1.0294x over previous
import functools

import jax
import jax.numpy as jnp
from jax import lax
from jax.experimental import pallas as pl
from jax.experimental.pallas import tpu as pltpu
from jax.experimental.pallas import tpu_sc as plsc

N = 10000
E = 160000
IN_DIM = 256
OUT_DIM = 256
NUM_HEADS = 4
HEAD_DIM = OUT_DIM // NUM_HEADS
EPS = 1e-5

NC = 2
NS = 16
L = 16
HALF = OUT_DIM // NC
EPT = E // NS
K = 128
NCHUNK = 80
EPT_P = NCHUNK * K
BLK = 16
NBLK = NCHUNK // BLK
ACC_ROWS = 10240
RPT = ACC_ROWS // NS
PAD_DST = ACC_ROWS - 1


def _mm_body(x_ref, w_ref, o_ref):
    o_ref[0] = jnp.dot(x_ref[...], w_ref[...],
                       preferred_element_type=jnp.float32)


def _matmul_split(x, w_cat):
    RB = 2000
    return pl.pallas_call(
        _mm_body,
        grid=(N // RB, NC),
        in_specs=[
            pl.BlockSpec((RB, IN_DIM), lambda i, j: (i, 0)),
            pl.BlockSpec((IN_DIM, HALF), lambda i, j: (0, j)),
        ],
        out_specs=pl.BlockSpec((1, RB, HALF), lambda i, j: (j, i, 0)),
        out_shape=jax.ShapeDtypeStruct((NC, N, HALF), jnp.float32),
    )(x, w_cat)


def _sc_body(h_hbm, src_hbm, dst_hbm, val_hbm, zeros_hbm, out_hbm,
             srcA, srcB, dstA, dstB, valA, valB, rows0, rows1, acc,
             gsem0, gsem1, ssem0, ssem1):
    c = lax.axis_index("c")
    s = lax.axis_index("s")
    hview = h_hbm.at[c]

    def load_block(b, srcb, dstb, valb):
        pltpu.sync_copy(src_hbm.at[s].at[pl.ds(b * BLK, BLK)], srcb)
        pltpu.sync_copy(dst_hbm.at[s].at[pl.ds(b * BLK, BLK)], dstb)
        pltpu.sync_copy(val_hbm.at[s].at[pl.ds(b * BLK, BLK)], valb)

    def gather_start(srcb, j, rows, sem):
        pltpu.async_copy(hview.at[srcb.at[j]], rows, sem)

    def gather_wait(srcb, rows, sem):
        pltpu.make_async_copy(hview.at[srcb.at[0]], rows, sem).wait()

    def scat_start(dstb, j, rows, sem):
        pltpu.async_copy(rows, acc.at[dstb.at[j]], sem, add=True)

    def scat_wait(dstb, rows, sem):
        pltpu.make_async_copy(rows, acc.at[dstb.at[0]], sem).wait()

    def scale(valb, j, rows):
        def group(g, cc):
            vv = valb[j, pl.ds(g * L, L)]
            for i in range(L):
                vsp = jnp.full((L,), vv[i], jnp.float32)
                for ci in range(HALF // L):
                    sl = pl.ds(ci * L, L)
                    rows[g * L + i, sl] = rows[g * L + i, sl] * vsp
            return cc

        lax.fori_loop(0, K // L, group, 0)

    bufs = [(srcA, dstA, valA), (srcB, dstB, valB)]
    load_block(0, *bufs[0])
    gather_start(bufs[0][0], 0, rows0, gsem0)

    pltpu.sync_copy(zeros_hbm, acc.at[pl.ds(s * RPT, RPT)])
    plsc.subcore_barrier()

    for b in range(NBLK):
        srcb, dstb, valb = bufs[b % 2]

        def pair(j2, cc, srcb=srcb, dstb=dstb, valb=valb, first=(b == 0)):
            j = 2 * j2

            def wait_prev():
                scat_wait(dstb, rows1, ssem1)

            if first:
                pl.when(j2 > 0)(wait_prev)
            else:
                wait_prev()

            gather_start(srcb, j + 1, rows1, gsem1)
            gather_wait(srcb, rows0, gsem0)
            scale(valb, j, rows0)
            scat_start(dstb, j, rows0, ssem0)
            gather_wait(srcb, rows1, gsem1)
            scale(valb, j + 1, rows1)
            scat_wait(dstb, rows0, ssem0)

            @pl.when(j2 < BLK // 2 - 1)
            def _():
                gather_start(srcb, j + 2, rows0, gsem0)

            scat_start(dstb, j + 1, rows1, ssem1)
            return cc

        lax.fori_loop(0, BLK // 2, pair, 0)

        if b + 1 < NBLK:
            load_block(b + 1, *bufs[(b + 1) % 2])
            gather_start(bufs[(b + 1) % 2][0], 0, rows0, gsem0)

    scat_wait(bufs[(NBLK - 1) % 2][1], rows1, ssem1)
    plsc.subcore_barrier()

    pltpu.sync_copy(acc.at[pl.ds(s * RPT, RPT)],
                    out_hbm.at[c].at[pl.ds(s * RPT, RPT)])


def _sc_scatter(h_split, src, dst, val, zeros):
    mesh = plsc.VectorSubcoreMesh(core_axis_name="c", subcore_axis_name="s")
    return pl.kernel(
        _sc_body,
        out_type=jax.ShapeDtypeStruct((NC, ACC_ROWS, HALF), jnp.float32),
        mesh=mesh,
        scratch_types=[
            pltpu.VMEM((BLK, K), jnp.int32),
            pltpu.VMEM((BLK, K), jnp.int32),
            pltpu.VMEM((BLK, K), jnp.int32),
            pltpu.VMEM((BLK, K), jnp.int32),
            pltpu.VMEM((BLK, K), jnp.float32),
            pltpu.VMEM((BLK, K), jnp.float32),
            pltpu.VMEM((K, HALF), jnp.float32),
            pltpu.VMEM((K, HALF), jnp.float32),
            pltpu.VMEM_SHARED((ACC_ROWS, HALF), jnp.float32),
            pltpu.SemaphoreType.DMA,
            pltpu.SemaphoreType.DMA,
            pltpu.SemaphoreType.DMA,
            pltpu.SemaphoreType.DMA,
        ],
    )(h_split, src, dst, val, zeros)


def _ln_rows(a, g_row, b_row):
    mu = jnp.mean(a, axis=-1, keepdims=True)
    xc = a - mu
    var = jnp.mean(xc * xc, axis=-1, keepdims=True)
    return xc * lax.rsqrt(var + EPS) * g_row + b_row


def _ep_body(agg_ref, bias_ref, lng_ref, lnb_ref, outg_ref, outb_ref, o_ref):
    full = jnp.concatenate([agg_ref[0], agg_ref[1]], axis=-1)
    full = full + bias_ref[...]
    heads = []
    for h in range(NUM_HEADS):
        sl = slice(h * HEAD_DIM, (h + 1) * HEAD_DIM)
        heads.append(_ln_rows(full[:, sl], lng_ref[:, sl], lnb_ref[:, sl]))
    normed = jnp.concatenate(heads, axis=-1)
    o_ref[...] = _ln_rows(normed, outg_ref[...], outb_ref[...])


def _epilogue(agg_split, bias_row, lng_row, lnb_row, outg_row, outb_row):
    RB = 2000
    row_spec = pl.BlockSpec((1, OUT_DIM), lambda i: (0, 0))
    return pl.pallas_call(
        _ep_body,
        grid=(N // RB,),
        in_specs=[
            pl.BlockSpec((NC, RB, HALF), lambda i: (0, i, 0)),
            row_spec, row_spec, row_spec, row_spec, row_spec,
        ],
        out_specs=pl.BlockSpec((RB, OUT_DIM), lambda i: (i, 0)),
        out_shape=jax.ShapeDtypeStruct((N, OUT_DIM), jnp.float32),
    )(agg_split, bias_row, lng_row, lnb_row, outg_row, outb_row)


def kernel(x, adj_indices, adj_values, W, bias, ln_g, ln_b, out_g, out_b):
    w_cat = W.transpose(1, 0, 2).reshape(IN_DIM, OUT_DIM)
    h_split = _matmul_split(x, w_cat)

    pad = EPT_P - EPT
    dst = jnp.concatenate(
        [adj_indices[0].reshape(NS, EPT),
         jnp.full((NS, pad), PAD_DST, jnp.int32)], axis=1).reshape(NS, NCHUNK, K)
    src = jnp.concatenate(
        [adj_indices[1].reshape(NS, EPT),
         jnp.zeros((NS, pad), jnp.int32)], axis=1).reshape(NS, NCHUNK, K)
    val = jnp.concatenate(
        [adj_values.reshape(NS, EPT),
         jnp.zeros((NS, pad), jnp.float32)], axis=1).reshape(NS, NCHUNK, K)
    zeros = jnp.zeros((RPT, HALF), jnp.float32)
    agg_split = _sc_scatter(h_split, src, dst, val, zeros)

    bias_row = bias.reshape(1, OUT_DIM)
    lng_row = ln_g.reshape(1, OUT_DIM)
    lnb_row = ln_b.reshape(1, OUT_DIM)
    outg_row = out_g.reshape(1, OUT_DIM)
    outb_row = out_b.reshape(1, OUT_DIM)
    return _epilogue(agg_split, bias_row, lng_row, lnb_row, outg_row, outb_row)


# speedup vs baseline: 1.0565x; 1.0263x over previous
import functools

import jax
import jax.numpy as jnp
from jax import lax
from jax.experimental import pallas as pl
from jax.experimental.pallas import tpu as pltpu
from jax.experimental.pallas import tpu_sc as plsc

N = 10000
E = 160000
IN_DIM = 256
OUT_DIM = 256
NUM_HEADS = 4
HEAD_DIM = OUT_DIM // NUM_HEADS
EPS = 1e-5

NC = 2
NS = 16
L = 16
HALF = OUT_DIM // NC
EPT = E // NS
K = 128
NCHUNK = 80
EPT_P = NCHUNK * K
BLK = 16
NBLK = NCHUNK // BLK
ACC_ROWS = 10240
RPT = ACC_ROWS // NS
PAD_DST = ACC_ROWS - 1


def _mm_body(x_ref, w_ref, o_ref):
    o_ref[0] = jnp.dot(x_ref[...], w_ref[...],
                       preferred_element_type=jnp.float32)


def _matmul_split(x, w_cat):
    RB = 2000
    return pl.pallas_call(
        _mm_body,
        grid=(N // RB, NC),
        in_specs=[
            pl.BlockSpec((RB, IN_DIM), lambda i, j: (i, 0)),
            pl.BlockSpec((IN_DIM, HALF), lambda i, j: (0, j)),
        ],
        out_specs=pl.BlockSpec((1, RB, HALF), lambda i, j: (j, i, 0)),
        out_shape=jax.ShapeDtypeStruct((NC, N, HALF), jnp.float32),
    )(x, w_cat)


def _sc_body(h_hbm, src_hbm, dst_hbm, val_hbm, zeros_hbm, out_hbm,
             srcA, srcB, dstA, dstB, valA, valB, rows0, rows1, acc,
             gsem0, gsem1, ssem0, ssem1):
    c = lax.axis_index("c")
    s = lax.axis_index("s")
    hview = h_hbm.at[c]

    def load_block(b, srcb, dstb, valb):
        pltpu.sync_copy(src_hbm.at[s].at[pl.ds(b * BLK, BLK)], srcb)
        pltpu.sync_copy(dst_hbm.at[s].at[pl.ds(b * BLK, BLK)], dstb)
        pltpu.sync_copy(val_hbm.at[s].at[pl.ds(b * BLK, BLK)], valb)

    def gather_start(srcb, j, rows, sem):
        pltpu.async_copy(hview.at[srcb.at[j]], rows, sem)

    def gather_wait(srcb, rows, sem):
        pltpu.make_async_copy(hview.at[srcb.at[0]], rows, sem).wait()

    def scat_start(dstb, j, rows, sem):
        pltpu.async_copy(rows, acc.at[dstb.at[j]], sem, add=True)

    def scat_wait(dstb, rows, sem):
        pltpu.make_async_copy(rows, acc.at[dstb.at[0]], sem).wait()

    def scale(valb, j, rows):
        def group(g, cc):
            vv = valb[j, pl.ds(g * L, L)]
            for i in range(L):
                vsp = jnp.full((L,), vv[i], jnp.float32)
                for ci in range(HALF // L):
                    sl = pl.ds(ci * L, L)
                    rows[g * L + i, sl] = rows[g * L + i, sl] * vsp
            return cc

        lax.fori_loop(0, K // L, group, 0)

    bufs = [(srcA, dstA, valA), (srcB, dstB, valB)]
    load_block(0, *bufs[0])
    gather_start(bufs[0][0], 0, rows0, gsem0)

    pltpu.sync_copy(zeros_hbm, acc.at[pl.ds(s * RPT, RPT)])
    plsc.subcore_barrier()

    for b in range(NBLK):
        srcb, dstb, valb = bufs[b % 2]

        def pair(j2, cc, srcb=srcb, dstb=dstb, valb=valb, first=(b == 0)):
            j = 2 * j2

            def wait_prev():
                scat_wait(dstb, rows1, ssem1)

            if first:
                pl.when(j2 > 0)(wait_prev)
            else:
                wait_prev()

            gather_start(srcb, j + 1, rows1, gsem1)
            gather_wait(srcb, rows0, gsem0)
            scale(valb, j, rows0)
            scat_start(dstb, j, rows0, ssem0)
            gather_wait(srcb, rows1, gsem1)
            scale(valb, j + 1, rows1)
            scat_wait(dstb, rows0, ssem0)

            @pl.when(j2 < BLK // 2 - 1)
            def _():
                gather_start(srcb, j + 2, rows0, gsem0)

            scat_start(dstb, j + 1, rows1, ssem1)
            return cc

        lax.fori_loop(0, BLK // 2, pair, 0)

        if b + 1 < NBLK:
            load_block(b + 1, *bufs[(b + 1) % 2])
            gather_start(bufs[(b + 1) % 2][0], 0, rows0, gsem0)

    scat_wait(bufs[(NBLK - 1) % 2][1], rows1, ssem1)
    plsc.subcore_barrier()

    pltpu.sync_copy(acc.at[pl.ds(s * RPT, RPT)],
                    out_hbm.at[c].at[pl.ds(s * RPT, RPT)])


def _sc_scatter(h_split, src, dst, val, zeros):
    mesh = plsc.VectorSubcoreMesh(core_axis_name="c", subcore_axis_name="s")
    return pl.kernel(
        _sc_body,
        out_type=jax.ShapeDtypeStruct((NC, ACC_ROWS, HALF), jnp.float32),
        mesh=mesh,
        scratch_types=[
            pltpu.VMEM((BLK, K), jnp.int32),
            pltpu.VMEM((BLK, K), jnp.int32),
            pltpu.VMEM((BLK, K), jnp.int32),
            pltpu.VMEM((BLK, K), jnp.int32),
            pltpu.VMEM((BLK, K), jnp.float32),
            pltpu.VMEM((BLK, K), jnp.float32),
            pltpu.VMEM((K, HALF), jnp.float32),
            pltpu.VMEM((K, HALF), jnp.float32),
            pltpu.VMEM_SHARED((ACC_ROWS, HALF), jnp.float32),
            pltpu.SemaphoreType.DMA,
            pltpu.SemaphoreType.DMA,
            pltpu.SemaphoreType.DMA,
            pltpu.SemaphoreType.DMA,
        ],
    )(h_split, src, dst, val, zeros)


def _ln_rows(a, g_row, b_row):
    mu = jnp.mean(a, axis=-1, keepdims=True)
    xc = a - mu
    var = jnp.mean(xc * xc, axis=-1, keepdims=True)
    return xc * lax.rsqrt(var + EPS) * g_row + b_row


def _ep_body(agg_ref, bias_ref, lng_ref, lnb_ref, outg_ref, outb_ref,
             md_ref, mup_ref, o_ref):
    full = jnp.concatenate([agg_ref[0], agg_ref[1]], axis=-1)
    full = full + bias_ref[...]
    mu_b = jnp.dot(jnp.dot(full, md_ref[...],
                           preferred_element_type=jnp.float32),
                   mup_ref[...], preferred_element_type=jnp.float32)
    xc = full - mu_b
    var_b = jnp.dot(jnp.dot(xc * xc, md_ref[...],
                            preferred_element_type=jnp.float32),
                    mup_ref[...], preferred_element_type=jnp.float32)
    normed = xc * lax.rsqrt(var_b + EPS) * lng_ref[...] + lnb_ref[...]
    o_ref[...] = _ln_rows(normed, outg_ref[...], outb_ref[...])


def _epilogue(agg_split, bias_row, lng_row, lnb_row, outg_row, outb_row):
    RB = 2000
    row_spec = pl.BlockSpec((1, OUT_DIM), lambda i: (0, 0))
    md = jnp.kron(jnp.eye(NUM_HEADS, dtype=jnp.float32),
                  jnp.ones((HEAD_DIM, 1), jnp.float32)) / HEAD_DIM
    mup = jnp.kron(jnp.eye(NUM_HEADS, dtype=jnp.float32),
                   jnp.ones((1, HEAD_DIM), jnp.float32))
    return pl.pallas_call(
        _ep_body,
        grid=(N // RB,),
        in_specs=[
            pl.BlockSpec((NC, RB, HALF), lambda i: (0, i, 0)),
            row_spec, row_spec, row_spec, row_spec, row_spec,
            pl.BlockSpec((OUT_DIM, NUM_HEADS), lambda i: (0, 0)),
            pl.BlockSpec((NUM_HEADS, OUT_DIM), lambda i: (0, 0)),
        ],
        out_specs=pl.BlockSpec((RB, OUT_DIM), lambda i: (i, 0)),
        out_shape=jax.ShapeDtypeStruct((N, OUT_DIM), jnp.float32),
    )(agg_split, bias_row, lng_row, lnb_row, outg_row, outb_row, md, mup)


def kernel(x, adj_indices, adj_values, W, bias, ln_g, ln_b, out_g, out_b):
    w_cat = W.transpose(1, 0, 2).reshape(IN_DIM, OUT_DIM)
    h_split = _matmul_split(x, w_cat)

    pad = EPT_P - EPT
    dst = jnp.concatenate(
        [adj_indices[0].reshape(NS, EPT),
         jnp.full((NS, pad), PAD_DST, jnp.int32)], axis=1).reshape(NS, NCHUNK, K)
    src = jnp.concatenate(
        [adj_indices[1].reshape(NS, EPT),
         jnp.zeros((NS, pad), jnp.int32)], axis=1).reshape(NS, NCHUNK, K)
    val = jnp.concatenate(
        [adj_values.reshape(NS, EPT),
         jnp.zeros((NS, pad), jnp.float32)], axis=1).reshape(NS, NCHUNK, K)
    zeros = jnp.zeros((RPT, HALF), jnp.float32)
    agg_split = _sc_scatter(h_split, src, dst, val, zeros)

    bias_row = bias.reshape(1, OUT_DIM)
    lng_row = ln_g.reshape(1, OUT_DIM)
    lnb_row = ln_b.reshape(1, OUT_DIM)
    outg_row = out_g.reshape(1, OUT_DIM)
    outb_row = out_b.reshape(1, OUT_DIM)
    return _epilogue(agg_split, bias_row, lng_row, lnb_row, outg_row, outb_row)
